# baseline (speedup 1.0000x reference)
import functools

import jax
import jax.numpy as jnp
from jax import lax
from jax.experimental import pallas as pl
from jax.experimental.pallas import tpu as pltpu
from jax.experimental.pallas import tpu_sc as plsc

_NC = 2
_NS = 16
_NW = _NC * _NS

_CH = 128


def _gather_body(n_ch, batches_per_w, k, idx_hbm, x_hbm, out_hbm,
                 idx_v, buf, sem):
    wid = lax.axis_index("s") * _NC + lax.axis_index("c")
    pltpu.sync_copy(idx_hbm.at[wid], idx_v)
    ch_per_b = n_ch // batches_per_w
    for half in range(batches_per_w):
        b = wid * batches_per_w + half
        for cc in range(ch_per_b):
            ci = half * ch_per_b + cc
            pltpu.async_copy(x_hbm.at[b].at[idx_v.at[ci]], buf, sem).wait()
            pltpu.sync_copy(buf, out_hbm.at[b, pl.ds(cc * _CH, _CH)])


def kernel(topk_idx, x):
    B, N, D = x.shape
    Bi, k = topk_idx.shape
    assert Bi == B
    total = B * k
    rows_per_w = total // _NW
    batches_per_w = B // _NW
    n_ch = rows_per_w // _CH

    idx = topk_idx.astype(jnp.int32).reshape(_NW, n_ch, _CH)

    mesh = plsc.VectorSubcoreMesh(core_axis_name="c", subcore_axis_name="s")
    run = pl.kernel(
        functools.partial(_gather_body, n_ch, batches_per_w, k),
        mesh=mesh,
        out_type=jax.ShapeDtypeStruct((B, k, D), x.dtype),
        scratch_types=[
            pltpu.VMEM((n_ch, _CH), jnp.int32),
            pltpu.VMEM((_CH, D), jnp.float32),
            pltpu.SemaphoreType.DMA,
        ],
        compiler_params=pltpu.CompilerParams(use_tc_tiling_on_sc=False),
    )
    return run(idx, x)


# speedup vs baseline: 1.5113x; 1.5113x over previous
import functools

import jax
import jax.numpy as jnp
from jax import lax
from jax.experimental import pallas as pl
from jax.experimental.pallas import tpu as pltpu
from jax.experimental.pallas import tpu_sc as plsc

_NC = 2
_NS = 16
_NW = _NC * _NS
_SUB = 1024
_CH = 256


def _body(k, idx_hbm, x_hbm, out_hbm, idx_v, rowbuf, sem):
    wid = lax.axis_index("s") * _NC + lax.axis_index("c")
    n_sub_per_b = k // _SUB
    for half in range(2):
        b = wid * 2 + half
        for sub in range(n_sub_per_b):
            pltpu.sync_copy(idx_hbm.at[wid, half * n_sub_per_b + sub], idx_v)
            for ch in range(_SUB // _CH):
                def issue(g, carry):
                    vec = idx_v[pl.ds(ch * _CH + g * 16, 16)]
                    for l in range(16):
                        pltpu.async_copy(
                            x_hbm.at[b, vec[l]], rowbuf.at[g * 16 + l], sem)
                    return carry
                lax.fori_loop(0, _CH // 16, issue, 0)
                pltpu.make_async_copy(
                    x_hbm.at[b, pl.ds(0, _CH)], rowbuf, sem).wait()
                pltpu.sync_copy(
                    rowbuf,
                    out_hbm.at[b, pl.ds((sub * (_SUB // _CH) + ch) * _CH, _CH)])


def kernel(topk_idx, x):
    B, N, D = x.shape
    k = topk_idx.shape[1]

    idx = topk_idx.astype(jnp.int32).reshape(_NW, (B * k) // (_NW * _SUB), _SUB)

    mesh = plsc.VectorSubcoreMesh(core_axis_name="c", subcore_axis_name="s")
    run = pl.kernel(
        functools.partial(_body, k),
        mesh=mesh,
        out_type=jax.ShapeDtypeStruct((B, k, D), x.dtype),
        scratch_types=[
            pltpu.VMEM((_SUB,), jnp.int32),
            pltpu.VMEM((_CH, D), jnp.float32),
            pltpu.SemaphoreType.DMA,
        ],
    )
    return run(idx, x)
